# initial kernel scaffold (unmeasured)
import jax
import jax.numpy as jnp
from jax import lax
from jax.experimental import pallas as pl
from jax.experimental.pallas import tpu as pltpu

N_DEV = 8
B = 2
SQ = 512
D_MODEL = 768
DH = 64
HQ_LOC = 8
D_LOC = HQ_LOC * DH
CHUNK = SQ // N_DEV
N_STEPS = 2 * (N_DEV - 1)


def kernel(x, Wq, K_ext, V_ext, Wo):
    my = lax.axis_index("i")
    wq_loc = lax.dynamic_slice_in_dim(Wq, my * D_LOC, D_LOC, axis=1)
    wo_loc = lax.dynamic_slice_in_dim(Wo, my * D_LOC, D_LOC, axis=0)
    k_t = jnp.transpose(K_ext, (0, 2, 1, 3))
    v_t = jnp.transpose(V_ext, (0, 2, 1, 3))

    def body(x_ref, wq_ref, k_ref, v_ref, wo_ref, out_ref,
             acc_ref, ctx_ref, comm_ref, send_sems, recv_sems):
        my_pos = lax.axis_index("i")
        right = (my_pos + 1) % N_DEV

        r = lax.broadcasted_iota(jnp.int32, (SQ, SQ), 0)
        c = lax.broadcasted_iota(jnp.int32, (SQ, SQ), 1)
        mask = ((r // DH) % 4) == ((c // DH) % 4)

        wq = wq_ref[:].astype(jnp.bfloat16)
        wo = wo_ref[:].astype(jnp.bfloat16)
        for b in range(B):
            xb = x_ref[b].astype(jnp.bfloat16)
            qb = lax.dot(xb, wq, preferred_element_type=jnp.float32)
            qb = qb.astype(jnp.bfloat16)
            for h in range(HQ_LOC):
                qh = qb[:, h * DH:(h + 1) * DH]
                kh = k_ref[b, h].astype(jnp.bfloat16)
                s = lax.dot_general(
                    qh, kh, (((1,), (1,)), ((), ())),
                    preferred_element_type=jnp.float32) * 0.125
                s = jnp.where(mask, s, -1e9)
                s = s - jnp.max(s, axis=1, keepdims=True)
                w = jnp.exp(s)
                w = w / jnp.sum(w, axis=1, keepdims=True)
                vh = v_ref[b, h].astype(jnp.bfloat16)
                ctx_ref[b, :, h * DH:(h + 1) * DH] = lax.dot(
                    w.astype(jnp.bfloat16), vh,
                    preferred_element_type=jnp.float32).astype(jnp.bfloat16)
            acc_ref[b] = lax.dot(ctx_ref[b], wo,
                                 preferred_element_type=jnp.float32)

        for h in range(N_DEV - 1):
            send_idx = (my_pos - h) % N_DEV
            recv_idx = (my_pos - h - 1) % N_DEV
            rdma = pltpu.make_async_remote_copy(
                src_ref=acc_ref.at[:, pl.ds(send_idx * CHUNK, CHUNK), :],
                dst_ref=comm_ref.at[h],
                send_sem=send_sems.at[h],
                recv_sem=recv_sems.at[h],
                device_id=(right,),
                device_id_type=pl.DeviceIdType.MESH,
            )
            rdma.start()
            rdma.wait()
            cur = acc_ref[:, pl.ds(recv_idx * CHUNK, CHUNK), :]
            acc_ref[:, pl.ds(recv_idx * CHUNK, CHUNK), :] = cur + comm_ref[h]

        own = (my_pos + 1) % N_DEV
        out_ref[:, pl.ds(own * CHUNK, CHUNK), :] = (
            acc_ref[:, pl.ds(own * CHUNK, CHUNK), :])

        for g in range(N_DEV - 1):
            h = (N_DEV - 1) + g
            send_idx = (my_pos + 1 - g) % N_DEV
            recv_idx = (my_pos - g) % N_DEV
            rdma = pltpu.make_async_remote_copy(
                src_ref=out_ref.at[:, pl.ds(send_idx * CHUNK, CHUNK), :],
                dst_ref=comm_ref.at[h],
                send_sem=send_sems.at[h],
                recv_sem=recv_sems.at[h],
                device_id=(right,),
                device_id_type=pl.DeviceIdType.MESH,
            )
            rdma.start()
            rdma.wait()
            out_ref[:, pl.ds(recv_idx * CHUNK, CHUNK), :] = comm_ref[h]

    return pl.pallas_call(
        body,
        out_shape=jax.ShapeDtypeStruct((B, SQ, D_MODEL), jnp.float32),
        in_specs=[pl.BlockSpec(memory_space=pltpu.VMEM)] * 5,
        out_specs=pl.BlockSpec(memory_space=pltpu.VMEM),
        scratch_shapes=[
            pltpu.VMEM((B, SQ, D_MODEL), jnp.float32),
            pltpu.VMEM((B, SQ, D_LOC), jnp.bfloat16),
            pltpu.VMEM((N_STEPS, B, CHUNK, D_MODEL), jnp.float32),
            pltpu.SemaphoreType.DMA((N_STEPS,)),
            pltpu.SemaphoreType.DMA((N_STEPS,)),
        ],
        compiler_params=pltpu.CompilerParams(collective_id=0),
    )(x, wq_loc, k_t, v_t, wo_loc)


# baseline (device time: 109416 ns/iter reference)
import jax
import jax.numpy as jnp
from jax import lax
from jax.experimental import pallas as pl
from jax.experimental.pallas import tpu as pltpu

N_DEV = 8
B = 2
SQ = 512
D_MODEL = 768
DH = 64
HQ_LOC = 8
D_LOC = HQ_LOC * DH
CHUNK = SQ // N_DEV
N_STEPS = 2 * (N_DEV - 1)


def kernel(x, Wq, K_ext, V_ext, Wo):
    my = lax.axis_index("i")
    wq_loc = lax.dynamic_slice_in_dim(Wq, my * D_LOC, D_LOC, axis=1)
    wo_loc = lax.dynamic_slice_in_dim(Wo, my * D_LOC, D_LOC, axis=0)
    k_t = jnp.transpose(K_ext, (0, 2, 1, 3))
    v_t = jnp.transpose(V_ext, (0, 2, 1, 3))

    def body(x_ref, wq_ref, k_ref, v_ref, wo_ref, out_ref,
             acc_ref, ctx_ref, comm_ref, send_sems, recv_sems):
        my_pos = lax.axis_index("i")
        right = (my_pos + 1) % N_DEV

        r = lax.broadcasted_iota(jnp.int32, (SQ, SQ), 0)
        c = lax.broadcasted_iota(jnp.int32, (SQ, SQ), 1)
        mask = ((r // DH) % 4) == ((c // DH) % 4)

        wq = wq_ref[:].astype(jnp.bfloat16)
        wo = wo_ref[:].astype(jnp.bfloat16)
        for b in range(B):
            xb = x_ref[b].astype(jnp.bfloat16)
            qb = lax.dot(xb, wq, preferred_element_type=jnp.float32)
            qb = qb.astype(jnp.bfloat16)
            for h in range(HQ_LOC):
                qh = qb[:, h * DH:(h + 1) * DH]
                kh = k_ref[b, h].astype(jnp.bfloat16)
                s = lax.dot_general(
                    qh, kh, (((1,), (1,)), ((), ())),
                    preferred_element_type=jnp.float32) * 0.125
                s = jnp.where(mask, s, -1e9)
                s = s - jnp.max(s, axis=1, keepdims=True)
                w = jnp.exp(s)
                w = w / jnp.sum(w, axis=1, keepdims=True)
                vh = v_ref[b, h].astype(jnp.bfloat16)
                ctx_ref[b, :, h * DH:(h + 1) * DH] = lax.dot(
                    w.astype(jnp.bfloat16), vh,
                    preferred_element_type=jnp.float32).astype(jnp.bfloat16)
            acc_ref[b] = lax.dot(ctx_ref[b], wo,
                                 preferred_element_type=jnp.float32)

        for h in range(N_DEV - 1):
            send_idx = (my_pos - h) % N_DEV
            recv_idx = (my_pos - h - 1) % N_DEV
            rdma = pltpu.make_async_remote_copy(
                src_ref=acc_ref.at[:, pl.ds(send_idx * CHUNK, CHUNK), :],
                dst_ref=comm_ref.at[h],
                send_sem=send_sems.at[h],
                recv_sem=recv_sems.at[h],
                device_id=(right,),
                device_id_type=pl.DeviceIdType.MESH,
            )
            rdma.start()
            rdma.wait()
            cur = acc_ref[:, pl.ds(recv_idx * CHUNK, CHUNK), :]
            acc_ref[:, pl.ds(recv_idx * CHUNK, CHUNK), :] = cur + comm_ref[h]

        own = (my_pos + 1) % N_DEV
        out_ref[:, pl.ds(own * CHUNK, CHUNK), :] = (
            acc_ref[:, pl.ds(own * CHUNK, CHUNK), :])

        for g in range(N_DEV - 1):
            h = (N_DEV - 1) + g
            send_idx = (my_pos + 1 - g) % N_DEV
            recv_idx = (my_pos - g) % N_DEV
            rdma = pltpu.make_async_remote_copy(
                src_ref=out_ref.at[:, pl.ds(send_idx * CHUNK, CHUNK), :],
                dst_ref=comm_ref.at[h],
                send_sem=send_sems.at[h],
                recv_sem=recv_sems.at[h],
                device_id=(right,),
                device_id_type=pl.DeviceIdType.MESH,
            )
            rdma.start()
            rdma.wait()
            out_ref[:, pl.ds(recv_idx * CHUNK, CHUNK), :] = comm_ref[h]

    return pl.pallas_call(
        body,
        out_shape=jax.ShapeDtypeStruct((B, SQ, D_MODEL), jnp.float32),
        in_specs=[pl.BlockSpec(memory_space=pltpu.VMEM)] * 5,
        out_specs=pl.BlockSpec(memory_space=pltpu.VMEM),
        scratch_shapes=[
            pltpu.VMEM((B, SQ, D_MODEL), jnp.float32),
            pltpu.VMEM((B, SQ, D_LOC), jnp.bfloat16),
            pltpu.VMEM((N_STEPS, B, CHUNK, D_MODEL), jnp.float32),
            pltpu.SemaphoreType.DMA((N_STEPS,)),
            pltpu.SemaphoreType.DMA((N_STEPS,)),
        ],
    )(x, wq_loc, k_t, v_t, wo_loc)


# device time: 63434 ns/iter; 1.7249x vs baseline; 1.7249x over previous
import jax
import jax.numpy as jnp
from jax import lax
from jax.experimental import pallas as pl
from jax.experimental.pallas import tpu as pltpu

N_DEV = 8
B = 2
SQ = 512
D_MODEL = 768
DH = 64
HQ_LOC = 8
D_LOC = HQ_LOC * DH


def kernel(x, Wq, K_ext, V_ext, Wo):
    my = lax.axis_index("i")
    wq_loc = lax.dynamic_slice_in_dim(Wq, my * D_LOC, D_LOC, axis=1)
    wo_loc = lax.dynamic_slice_in_dim(Wo, my * D_LOC, D_LOC, axis=0)
    k_t = jnp.transpose(K_ext, (0, 2, 1, 3))
    v_t = jnp.transpose(V_ext, (0, 2, 1, 3))

    def body(x_ref, wq_ref, k_ref, v_ref, wo_ref, out_ref,
             acc_ref, ctx_ref, g_ref, r0_ref, r1_ref, r2_ref,
             send_sems, recv_sems):
        p = lax.axis_index("i")
        pz = (p + 4) % 8
        py = (p // 4) * 4 + 3 - (p % 4)
        px = p + 1 - 2 * (p % 2)
        my4 = (p // 4) * 256
        my2 = (p // 2) * 128
        my1 = p * 64

        r = lax.broadcasted_iota(jnp.int32, (SQ, SQ), 0)
        c = lax.broadcasted_iota(jnp.int32, (SQ, SQ), 1)
        mask = ((r // DH) % 4) == ((c // DH) % 4)

        wq = wq_ref[:].astype(jnp.bfloat16)
        wo = wo_ref[:].astype(jnp.bfloat16)
        for b in range(B):
            xb = x_ref[b].astype(jnp.bfloat16)
            qb = lax.dot(xb, wq, preferred_element_type=jnp.float32)
            qb = qb.astype(jnp.bfloat16)
            for h in range(HQ_LOC):
                qh = qb[:, h * DH:(h + 1) * DH]
                kh = k_ref[b, h].astype(jnp.bfloat16)
                s = lax.dot_general(
                    qh, kh, (((1,), (1,)), ((), ())),
                    preferred_element_type=jnp.float32) * 0.125
                s = jnp.where(mask, s, -1e9)
                s = s - jnp.max(s, axis=1, keepdims=True)
                w = jnp.exp(s)
                w = w / jnp.sum(w, axis=1, keepdims=True)
                vh = v_ref[b, h].astype(jnp.bfloat16)
                ctx_ref[b, :, h * DH:(h + 1) * DH] = lax.dot(
                    w.astype(jnp.bfloat16), vh,
                    preferred_element_type=jnp.float32).astype(jnp.bfloat16)
            acc_ref[b] = lax.dot(ctx_ref[b], wo,
                                 preferred_element_type=jnp.float32
                                 ).astype(jnp.bfloat16)

        def exchange(step, src, dst, partner):
            rdma = pltpu.make_async_remote_copy(
                src_ref=src, dst_ref=dst,
                send_sem=send_sems.at[step], recv_sem=recv_sems.at[step],
                device_id=(partner,), device_id_type=pl.DeviceIdType.MESH,
            )
            rdma.start()
            rdma.wait()

        exchange(0, acc_ref.at[:, pl.ds(256 - my4, 256), :], r0_ref, pz)
        acc_ref[:, pl.ds(my4, 256), :] += r0_ref[:, :, :]
        exchange(1, acc_ref.at[:, pl.ds((py // 2) * 128, 128), :], r1_ref, py)
        acc_ref[:, pl.ds(my2, 128), :] += r1_ref[:, :, :]
        exchange(2, acc_ref.at[:, pl.ds(px * 64, 64), :], r2_ref, px)
        acc_ref[:, pl.ds(my1, 64), :] += r2_ref[:, :, :]

        g_ref[:, pl.ds(my1, 64), :] = acc_ref[:, pl.ds(my1, 64), :]

        exchange(3, g_ref.at[:, pl.ds(my1, 64), :],
                 g_ref.at[:, pl.ds(my1, 64), :], px)
        exchange(4, g_ref.at[:, pl.ds(my2, 128), :],
                 g_ref.at[:, pl.ds(my2, 128), :], py)
        exchange(5, g_ref.at[:, pl.ds(my4, 256), :],
                 g_ref.at[:, pl.ds(my4, 256), :], pz)

        out_ref[:, :, :] = g_ref[:, :, :].astype(jnp.float32)

    return pl.pallas_call(
        body,
        out_shape=jax.ShapeDtypeStruct((B, SQ, D_MODEL), jnp.float32),
        in_specs=[pl.BlockSpec(memory_space=pltpu.VMEM)] * 5,
        out_specs=pl.BlockSpec(memory_space=pltpu.VMEM),
        scratch_shapes=[
            pltpu.VMEM((B, SQ, D_MODEL), jnp.bfloat16),
            pltpu.VMEM((B, SQ, D_LOC), jnp.bfloat16),
            pltpu.VMEM((B, SQ, D_MODEL), jnp.bfloat16),
            pltpu.VMEM((B, 256, D_MODEL), jnp.bfloat16),
            pltpu.VMEM((B, 128, D_MODEL), jnp.bfloat16),
            pltpu.VMEM((B, 64, D_MODEL), jnp.bfloat16),
            pltpu.SemaphoreType.DMA((6,)),
            pltpu.SemaphoreType.DMA((6,)),
        ],
    )(x, wq_loc, k_t, v_t, wo_loc)


# device time: 49848 ns/iter; 2.1950x vs baseline; 1.2725x over previous
import jax
import jax.numpy as jnp
from jax import lax
from jax.experimental import pallas as pl
from jax.experimental.pallas import tpu as pltpu

N_DEV = 8
B = 2
SQ = 512
D_MODEL = 768
DH = 64
HQ_LOC = 8
D_LOC = HQ_LOC * DH


def kernel(x, Wq, K_ext, V_ext, Wo):
    my = lax.axis_index("i")
    wq_loc = lax.dynamic_slice_in_dim(Wq, my * D_LOC, D_LOC, axis=1)
    wo_loc = lax.dynamic_slice_in_dim(Wo, my * D_LOC, D_LOC, axis=0)
    k_t = jnp.transpose(K_ext, (0, 2, 1, 3))
    v_t = jnp.transpose(V_ext, (0, 2, 1, 3))

    def body(x_ref, wq_ref, k_ref, v_ref, wo_ref, out_ref,
             acc_ref, ctx_ref, g_ref, r0_ref, r1_ref, r2_ref,
             send_sems, recv_sems):
        p = lax.axis_index("i")
        pz = (p + 4) % 8
        py = (p // 4) * 4 + 3 - (p % 4)
        px = p + 1 - 2 * (p % 2)

        r = lax.broadcasted_iota(jnp.int32, (SQ, SQ), 0)
        c = lax.broadcasted_iota(jnp.int32, (SQ, SQ), 1)
        mask = ((r // DH) % 4) == ((c // DH) % 4)

        wq = wq_ref[:].astype(jnp.bfloat16)
        wo = wo_ref[:].astype(jnp.bfloat16)
        for b in range(B):
            xb = x_ref[b].astype(jnp.bfloat16)
            qb = lax.dot(xb, wq, preferred_element_type=jnp.float32)
            qb = qb.astype(jnp.bfloat16)
            for h in range(HQ_LOC):
                qh = qb[:, h * DH:(h + 1) * DH]
                kh = k_ref[b, h].astype(jnp.bfloat16)
                s = lax.dot_general(
                    qh, kh, (((1,), (1,)), ((), ())),
                    preferred_element_type=jnp.float32) * 0.125
                s = jnp.where(mask, s, -1e9)
                s = s - jnp.max(s, axis=1, keepdims=True)
                w = jnp.exp(s)
                w = w / jnp.sum(w, axis=1, keepdims=True)
                vh = v_ref[b, h].astype(jnp.bfloat16)
                ctx_ref[b, :, h * DH:(h + 1) * DH] = lax.dot(
                    w.astype(jnp.bfloat16), vh,
                    preferred_element_type=jnp.float32).astype(jnp.bfloat16)
            acc_ref[b] = lax.dot(ctx_ref[b], wo,
                                 preferred_element_type=jnp.float32
                                 ).astype(jnp.bfloat16)

        partner = {"z": pz, "y": py, "x": px}
        side = {"z": p // 4, "y": (p % 4) // 2, "x": p % 2}
        orders = (("z", "y", "x"), ("y", "x", "z"))
        sizes = (256, 128, 64)
        stages = (r0_ref, r1_ref, r2_ref)
        keep, send_at = [], []
        for j in range(len(orders)):
            d0, d1, d2 = orders[j]
            k0 = 256 * side[d0]
            k1 = k0 + 128 * side[d1]
            k2 = k1 + 64 * side[d2]
            keep.append((k0, k1, k2))
            send_at.append((256 * (1 - side[d0]),
                            k0 + 128 * (1 - side[d1]),
                            k1 + 64 * (1 - side[d2])))

        for s in range(3):
            rdmas = []
            for j in range(len(orders)):
                rdma = pltpu.make_async_remote_copy(
                    src_ref=acc_ref.at[:, pl.ds(send_at[j][s], sizes[s]),
                                       j * 384:(j + 1) * 384],
                    dst_ref=stages[s].at[j],
                    send_sem=send_sems.at[s * 2 + j],
                    recv_sem=recv_sems.at[s * 2 + j],
                    device_id=(partner[orders[j][s]],),
                    device_id_type=pl.DeviceIdType.MESH,
                )
                rdma.start()
                rdmas.append(rdma)
            for j in range(len(orders)):
                rdmas[j].wait()
                cur = acc_ref[:, pl.ds(keep[j][s], sizes[s]),
                              j * 384:(j + 1) * 384]
                acc_ref[:, pl.ds(keep[j][s], sizes[s]),
                        j * 384:(j + 1) * 384] = cur + stages[s][j]

        for j in range(len(orders)):
            g_ref[:, pl.ds(keep[j][2], 64), j * 384:(j + 1) * 384] = (
                acc_ref[:, pl.ds(keep[j][2], 64), j * 384:(j + 1) * 384])

        for t in range(3):
            lvl = 2 - t
            rdmas = []
            for j in range(len(orders)):
                sl = (slice(None), pl.ds(keep[j][lvl], sizes[lvl]),
                      slice(j * 384, (j + 1) * 384))
                rdma = pltpu.make_async_remote_copy(
                    src_ref=g_ref.at[sl], dst_ref=g_ref.at[sl],
                    send_sem=send_sems.at[(3 + t) * 2 + j],
                    recv_sem=recv_sems.at[(3 + t) * 2 + j],
                    device_id=(partner[orders[j][lvl]],),
                    device_id_type=pl.DeviceIdType.MESH,
                )
                rdma.start()
                rdmas.append(rdma)
            for j in range(len(orders)):
                rdmas[j].wait()

        out_ref[:, :, :] = g_ref[:, :, :].astype(jnp.float32)

    return pl.pallas_call(
        body,
        out_shape=jax.ShapeDtypeStruct((B, SQ, D_MODEL), jnp.float32),
        in_specs=[pl.BlockSpec(memory_space=pltpu.VMEM)] * 5,
        out_specs=pl.BlockSpec(memory_space=pltpu.VMEM),
        scratch_shapes=[
            pltpu.VMEM((B, SQ, D_MODEL), jnp.bfloat16),
            pltpu.VMEM((B, SQ, D_LOC), jnp.bfloat16),
            pltpu.VMEM((B, SQ, D_MODEL), jnp.bfloat16),
            pltpu.VMEM((2, B, 256, 384), jnp.bfloat16),
            pltpu.VMEM((2, B, 128, 384), jnp.bfloat16),
            pltpu.VMEM((2, B, 64, 384), jnp.bfloat16),
            pltpu.SemaphoreType.DMA((12,)),
            pltpu.SemaphoreType.DMA((12,)),
        ],
    )(x, wq_loc, k_t, v_t, wo_loc)
